# baseline (speedup 1.0000x reference)
import jax
import jax.numpy as jnp
from jax.experimental import pallas as pl

_P = 0.1


def _expand_body(rand_ref, keep_ref):
    keep = (rand_ref[...] >= _P).astype(jnp.float32)
    keep_ref[...] = jnp.broadcast_to(keep, keep_ref.shape)


def _mul_body(keep_ref, mask_ref, x_ref, out_ref, xout_ref):
    out_ref[...] = mask_ref[...] * keep_ref[...][None]
    xout_ref[...] = x_ref[...]


def kernel(x, mask, rand_t):
    C, T, V = mask.shape
    CB = 4

    keep_tv = pl.pallas_call(
        _expand_body,
        out_shape=jax.ShapeDtypeStruct((T, V), jnp.float32),
    )(rand_t.reshape(T, 1))

    blk = pl.BlockSpec((CB, T, V), lambda i: (i, 0, 0))
    out, x_out = pl.pallas_call(
        _mul_body,
        grid=(C // CB,),
        in_specs=[
            pl.BlockSpec((T, V), lambda i: (0, 0)),
            blk,
            blk,
        ],
        out_specs=[blk, blk],
        out_shape=[
            jax.ShapeDtypeStruct((C, T, V), jnp.float32),
            jax.ShapeDtypeStruct((C, T, V), jnp.float32),
        ],
    )(keep_tv, mask, x)
    return (x_out, out)
